# Initial kernel scaffold; baseline (speedup 1.0000x reference)
import functools

import jax
import jax.numpy as jnp
from jax.experimental import pallas as pl
from jax.experimental.pallas import tpu as pltpu

N = 10000
E = 320000
D = 128
H = 128
G = 64
EPS = 1e-5

_PREC = jax.lax.Precision.HIGHEST


def _dot(a, b):
    return jax.lax.dot_general(a, b, (((1,), (0,)), ((), ())),
                               precision=_PREC,
                               preferred_element_type=jnp.float32)


def _pre_body(x_ref, w_ref, asrc_ref, adst_ref, ea_ref, aedge_ref,
              h_ref, s_ref, eaa_ref):
    x = x_ref[...]
    w = w_ref[...]
    h = _dot(x, w)
    h_ref[...] = h
    s_ref[0, :] = jnp.sum(h * asrc_ref[0, :][None, :], axis=1)
    s_ref[1, :] = jnp.sum(h * adst_ref[0, :][None, :], axis=1)
    eaa_ref[...] = ea_ref[...] * aedge_ref[0, 0]


def _pre(x, w, a_src, a_dst, ea, a_edge):
    return pl.pallas_call(
        _pre_body,
        out_shape=[
            jax.ShapeDtypeStruct((N, D), jnp.float32),
            jax.ShapeDtypeStruct((2, N), jnp.float32),
            jax.ShapeDtypeStruct((E, 1), jnp.float32),
        ],
    )(x, w, a_src.reshape(1, H), a_dst.reshape(1, H), ea,
      a_edge.reshape(1, 1))


def _instance_norm_in_kernel(y, bm):
    oh = (bm == jax.lax.broadcasted_iota(jnp.int32, (N, G), 1))
    oh = oh.astype(jnp.float32)
    ones = jnp.ones((N, 1), jnp.float32)
    cnt = jax.lax.dot_general(oh, ones, (((0,), (0,)), ((), ())),
                              precision=_PREC,
                              preferred_element_type=jnp.float32)
    cnt = jnp.maximum(cnt, 1.0)
    sums = jax.lax.dot_general(oh, y, (((0,), (0,)), ((), ())),
                               precision=_PREC,
                               preferred_element_type=jnp.float32)
    mean = sums / cnt
    xc = y - _dot(oh, mean)
    sq = jax.lax.dot_general(oh, xc * xc, (((0,), (0,)), ((), ())),
                             precision=_PREC,
                             preferred_element_type=jnp.float32)
    var = sq / cnt
    return xc / jnp.sqrt(_dot(oh, var) + EPS)


def _norm0_body(raw_ref, den_ref, bm_ref, o_ref):
    den = den_ref[0, :] + den_ref[1, :] + 1e-16
    y = raw_ref[...] / den[:, None]
    o_ref[...] = _instance_norm_in_kernel(y, bm_ref[...])


def _norm0(raw2, den2, bm):
    return pl.pallas_call(
        _norm0_body,
        out_shape=jax.ShapeDtypeStruct((N, H), jnp.float32),
    )(raw2, den2, bm.reshape(N, 1))


def _norm12_body(raw_ref, den_ref, bm_ref, xp_ref, wn_ref, wr_ref, b_ref,
                 o_ref):
    den = den_ref[0, :] + den_ref[1, :] + 1e-16
    agg = raw_ref[...] / den[:, None]
    z = _dot(agg, wn_ref[...]) + _dot(xp_ref[...], wr_ref[...]) \
        + b_ref[0, :][None, :]
    o_ref[...] = _instance_norm_in_kernel(z, bm_ref[...])


def _norm12(raw2, den2, bm, x_prev, wn, wr, b):
    return pl.pallas_call(
        _norm12_body,
        out_shape=jax.ShapeDtypeStruct((N, H), jnp.float32),
    )(raw2, den2, bm.reshape(N, 1), x_prev, wn, wr, b.reshape(1, H))


def kernel(x, ei, ea, batch_mask, W_egat, a_src, a_dst, a_edge,
           W1_root, W1_nbr, b1, W2_root, W2_nbr, b2):
    src, dst = ei[0], ei[1]

    h, s, eaa = _pre(x, W_egat, a_src, a_dst, ea, a_edge)
    eaa = eaa.reshape(-1)

    logit = s[0, src] + s[1, dst] + eaa
    e = jnp.maximum(logit, 0.2 * logit)
    ex = jnp.exp(e)
    den = jax.ops.segment_sum(ex, dst, num_segments=N)
    den2 = jnp.stack([den, jnp.zeros_like(den)])

    raw0 = jax.ops.segment_sum(ex[:, None] * h[src], dst, num_segments=N)
    x0 = _norm0(raw0, den2, batch_mask)

    wn1 = W1_nbr[:H] + W1_nbr[H:]
    wr1 = W1_root[:H] + W1_root[H:]
    raw1 = jax.ops.segment_sum(ex[:, None] * x0[src], dst, num_segments=N)
    x1 = _norm12(raw1, den2, batch_mask, x0, wn1, wr1, b1)

    wn2 = W2_nbr[:H] + W2_nbr[H:]
    wr2 = W2_root[:H] + W2_root[H:]
    raw2 = jax.ops.segment_sum(ex[:, None] * x1[src], dst, num_segments=N)
    x2 = _norm12(raw2, den2, batch_mask, x1, wn2, wr2, b2)

    return (x0, x1, x2)


# speedup vs baseline: 1.1292x; 1.1292x over previous
import functools

import jax
import jax.numpy as jnp
from jax.experimental import pallas as pl
from jax.experimental.pallas import tpu as pltpu

N = 10000
E = 320000
D = 128
H = 128
G = 64
EPS = 1e-5

_PREC = jax.lax.Precision.HIGHEST


def _dot(a, b):
    return jax.lax.dot_general(a, b, (((1,), (0,)), ((), ())),
                               precision=_PREC,
                               preferred_element_type=jnp.float32)


def _pre_body(x_ref, w_ref, asrc_ref, adst_ref, ea_ref, aedge_ref,
              h_ref, s_ref, eaa_ref):
    x = x_ref[...]
    w = w_ref[...]
    h = _dot(x, w)
    h_ref[...] = h
    s_ref[0, :] = jnp.sum(h * asrc_ref[0, :][None, :], axis=1)
    s_ref[1, :] = jnp.sum(h * adst_ref[0, :][None, :], axis=1)
    eaa_ref[...] = ea_ref[...] * aedge_ref[0, 0]


def _pre(x, w, a_src, a_dst, ea, a_edge):
    return pl.pallas_call(
        _pre_body,
        out_shape=[
            jax.ShapeDtypeStruct((N, D), jnp.float32),
            jax.ShapeDtypeStruct((2, N), jnp.float32),
            jax.ShapeDtypeStruct((E // 128, 128), jnp.float32),
        ],
    )(x, w, a_src.reshape(1, H), a_dst.reshape(1, H),
      ea.reshape(E // 128, 128), a_edge.reshape(1, 1))


def _instance_norm_in_kernel(y, bm):
    oh = (bm == jax.lax.broadcasted_iota(jnp.int32, (N, G), 1))
    oh = oh.astype(jnp.float32)
    ones = jnp.ones((N, 1), jnp.float32)
    cnt = jax.lax.dot_general(oh, ones, (((0,), (0,)), ((), ())),
                              precision=_PREC,
                              preferred_element_type=jnp.float32)
    cnt = jnp.maximum(cnt, 1.0)
    sums = jax.lax.dot_general(oh, y, (((0,), (0,)), ((), ())),
                               precision=_PREC,
                               preferred_element_type=jnp.float32)
    mean = sums / cnt
    xc = y - _dot(oh, mean)
    sq = jax.lax.dot_general(oh, xc * xc, (((0,), (0,)), ((), ())),
                             precision=_PREC,
                             preferred_element_type=jnp.float32)
    var = sq / cnt
    return xc / jnp.sqrt(_dot(oh, var) + EPS)


def _norm0_body(raw_ref, den_ref, bm_ref, o_ref):
    den = den_ref[0, :] + den_ref[1, :] + 1e-16
    y = raw_ref[...] / den[:, None]
    o_ref[...] = _instance_norm_in_kernel(y, bm_ref[...])


def _norm0(raw2, den2, bm):
    return pl.pallas_call(
        _norm0_body,
        out_shape=jax.ShapeDtypeStruct((N, H), jnp.float32),
    )(raw2, den2, bm.reshape(N, 1))


def _gc_body(raw_ref, den_ref, xp_ref, wn_ref, wr_ref, b_ref, z_ref):
    den = den_ref[0, :] + den_ref[1, :] + 1e-16
    agg = raw_ref[...] / den[:, None]
    z_ref[...] = _dot(agg, wn_ref[...]) + _dot(xp_ref[...], wr_ref[...]) \
        + b_ref[0, :][None, :]


def _norm12(raw2, den2, bm, x_prev, wn, wr, b):
    z = pl.pallas_call(
        _gc_body,
        out_shape=jax.ShapeDtypeStruct((N, H), jnp.float32),
    )(raw2, den2, x_prev, wn, wr, b.reshape(1, H))
    return pl.pallas_call(
        _norm_only_body,
        out_shape=jax.ShapeDtypeStruct((N, H), jnp.float32),
    )(z, bm.reshape(N, 1))


def _norm_only_body(z_ref, bm_ref, o_ref):
    o_ref[...] = _instance_norm_in_kernel(z_ref[...], bm_ref[...])


def kernel(x, ei, ea, batch_mask, W_egat, a_src, a_dst, a_edge,
           W1_root, W1_nbr, b1, W2_root, W2_nbr, b2):
    src, dst = ei[0], ei[1]

    h, s, eaa = _pre(x, W_egat, a_src, a_dst, ea, a_edge)
    eaa = eaa.reshape(-1)

    logit = s[0, src] + s[1, dst] + eaa
    e = jnp.maximum(logit, 0.2 * logit)
    ex = jnp.exp(e)
    den = jax.ops.segment_sum(ex, dst, num_segments=N)
    den2 = jnp.stack([den, jnp.zeros_like(den)])

    raw0 = jax.ops.segment_sum(ex[:, None] * h[src], dst, num_segments=N)
    x0 = _norm0(raw0, den2, batch_mask)

    wn1 = W1_nbr[:H] + W1_nbr[H:]
    wr1 = W1_root[:H] + W1_root[H:]
    raw1 = jax.ops.segment_sum(ex[:, None] * x0[src], dst, num_segments=N)
    x1 = _norm12(raw1, den2, batch_mask, x0, wn1, wr1, b1)

    wn2 = W2_nbr[:H] + W2_nbr[H:]
    wr2 = W2_root[:H] + W2_root[H:]
    raw2 = jax.ops.segment_sum(ex[:, None] * x1[src], dst, num_segments=N)
    x2 = _norm12(raw2, den2, batch_mask, x1, wn2, wr2, b2)

    return (x0, x1, x2)


# speedup vs baseline: 13.9207x; 12.3281x over previous
import dataclasses
import functools

import jax
import jax.numpy as jnp
from jax import lax
from jax.experimental import pallas as pl
from jax.experimental.pallas import tpu as pltpu
from jax.experimental.pallas import tpu_sc as plsc

N = 10000
E = 320000
D = 128
H = 128
G = 64
EPS = 1e-5

NP = 10240
RB = NP // 16
ECH = 2000
EPT = E // 32
GCH = 80
NCH = (E // 2) // 16 // GCH

_PREC = jax.lax.Precision.HIGHEST

_MESH = plsc.VectorSubcoreMesh(core_axis_name="c", subcore_axis_name="s",
                               num_cores=2, num_subcores=16)

_SC_CP = pltpu.CompilerParams()
if "needs_layout_passes" in pltpu.CompilerParams.__dataclass_fields__:
    _SC_CP = dataclasses.replace(_SC_CP, needs_layout_passes=False)


def _dot(a, b):
    return jax.lax.dot_general(a, b, (((1,), (0,)), ((), ())),
                               precision=_PREC,
                               preferred_element_type=jnp.float32)


def _pre_body(x_ref, w_ref, asrc_ref, adst_ref, ea_ref, aedge_ref,
              h_ref, s_ref, eaa_ref):
    x = x_ref[...]
    h = _dot(x, w_ref[...])
    h_ref[...] = h
    s_ref[0, :] = jnp.sum(h * asrc_ref[0, :][None, :], axis=1)
    s_ref[1, :] = jnp.sum(h * adst_ref[0, :][None, :], axis=1)
    eaa_ref[...] = ea_ref[...] * aedge_ref[0, 0]


def _pre(x, w, a_src, a_dst, ea, a_edge):
    return pl.pallas_call(
        _pre_body,
        out_shape=[
            jax.ShapeDtypeStruct((N, D), jnp.float32),
            jax.ShapeDtypeStruct((2, N), jnp.float32),
            jax.ShapeDtypeStruct((E // 128, 128), jnp.float32),
        ],
    )(x, w, a_src.reshape(1, H), a_dst.reshape(1, H),
      ea.reshape(E // 128, 128), a_edge.reshape(1, 1))


def _edge_softmax_sc(s2, src, dst, eaa):
    @functools.partial(
        pl.kernel,
        out_type=[jax.ShapeDtypeStruct((E,), jnp.float32),
                  jax.ShapeDtypeStruct((2, NP), jnp.float32)],
        mesh=_MESH,
        compiler_params=_SC_CP,
        scratch_types=[pltpu.VMEM((N,), jnp.float32),
                       pltpu.VMEM((N,), jnp.float32),
                       pltpu.VMEM((ECH,), jnp.int32),
                       pltpu.VMEM((ECH,), jnp.int32),
                       pltpu.VMEM((ECH,), jnp.float32),
                       pltpu.VMEM((ECH,), jnp.float32),
                       pltpu.VMEM((NP,), jnp.float32),
                       pltpu.VMEM((RB,), jnp.float32),
                       pltpu.VMEM((RB,), jnp.float32),
                       pltpu.VMEM_SHARED((16, NP), jnp.float32)],
    )
    def k(s_hbm, src_hbm, dst_hbm, eaa_hbm, ex_hbm, den_hbm,
          ssrc_v, sdst_v, src_v, dst_v, eaa_v, ex_v, den_v, comb_v, tmp_v,
          stage_sh):
        cid = lax.axis_index("c")
        sid = lax.axis_index("s")
        base = (cid * 16 + sid) * EPT
        pltpu.sync_copy(s_hbm.at[0], ssrc_v)
        pltpu.sync_copy(s_hbm.at[1], sdst_v)

        @pl.loop(0, NP, step=16)
        def _z(i):
            den_v[pl.ds(i, 16)] = jnp.zeros((16,), jnp.float32)

        @pl.loop(0, EPT, step=ECH)
        def _chunk(off):
            pltpu.sync_copy(src_hbm.at[pl.ds(base + off, ECH)], src_v)
            pltpu.sync_copy(dst_hbm.at[pl.ds(base + off, ECH)], dst_v)
            pltpu.sync_copy(eaa_hbm.at[pl.ds(base + off, ECH)], eaa_v)

            @pl.loop(0, ECH, step=16)
            def _grp(i):
                si = src_v[pl.ds(i, 16)]
                di = dst_v[pl.ds(i, 16)]
                a = plsc.load_gather(ssrc_v, [si])
                b = plsc.load_gather(sdst_v, [di])
                lg = a + b + eaa_v[pl.ds(i, 16)]
                e = jnp.maximum(lg, 0.2 * lg)
                exv = jnp.exp(e)
                ex_v[pl.ds(i, 16)] = exv
                plsc.addupdate_scatter(den_v, [di], exv)

            pltpu.sync_copy(ex_v, ex_hbm.at[pl.ds(base + off, ECH)])

        pltpu.sync_copy(den_v, stage_sh.at[sid])
        plsc.subcore_barrier()
        pltpu.sync_copy(stage_sh.at[0, pl.ds(sid * RB, RB)], comb_v)

        @pl.loop(1, 16)
        def _red(j):
            pltpu.sync_copy(stage_sh.at[j, pl.ds(sid * RB, RB)], tmp_v)

            @pl.loop(0, RB, step=16)
            def _add(i):
                comb_v[pl.ds(i, 16)] = (comb_v[pl.ds(i, 16)]
                                        + tmp_v[pl.ds(i, 16)])

        pltpu.sync_copy(comb_v, den_hbm.at[cid, pl.ds(sid * RB, RB)])

    return k(s2, src, dst, eaa)


def _spmm_sc(table, src, dst, ex):
    @functools.partial(
        pl.kernel,
        out_type=jax.ShapeDtypeStruct((2, NP, D), jnp.float32),
        mesh=_MESH,
        compiler_params=_SC_CP,
        scratch_types=[pltpu.VMEM((GCH,), jnp.int32),
                       pltpu.VMEM((GCH,), jnp.int32),
                       pltpu.VMEM((GCH,), jnp.int32),
                       pltpu.VMEM((GCH,), jnp.int32),
                       pltpu.VMEM((GCH,), jnp.float32),
                       pltpu.VMEM((GCH,), jnp.float32),
                       pltpu.VMEM((GCH, D), jnp.float32),
                       pltpu.VMEM((GCH, D), jnp.float32),
                       pltpu.VMEM_SHARED((NP, D), jnp.float32),
                       pltpu.SemaphoreType.DMA,
                       pltpu.SemaphoreType.DMA],
    )
    def k(tab_hbm, src_hbm, dst_hbm, ex_hbm, out_hbm,
          src_a, src_b, dst_a, dst_b, ex_a, ex_b, rows_a, rows_b, acc_sh,
          gsem_a, gsem_b):
        cid = lax.axis_index("c")
        sid = lax.axis_index("s")
        base = (cid * 16 + sid) * (NCH * GCH)

        @pl.loop(0, GCH)
        def _zr(r):
            for kk in range(D // 16):
                rows_a[r, pl.ds(kk * 16, 16)] = jnp.zeros((16,), jnp.float32)

        @pl.loop(0, RB, step=GCH)
        def _za(r):
            pltpu.sync_copy(rows_a, acc_sh.at[pl.ds(sid * RB + r, GCH)])

        plsc.subcore_barrier()

        _gdn = jax.lax.GatherDimensionNumbers(
            offset_dims=(), collapsed_slice_dims=(0,), start_index_map=(0,))

        def _scale(exbuf, rows):
            @pl.loop(0, GCH, step=16)
            def _g(i):
                ex16 = exbuf[pl.ds(i, 16)]
                for jj in range(16):
                    bex = jax.lax.gather(
                        ex16, jnp.full((16, 1), jj, jnp.int32), _gdn, (1,),
                        mode=jax.lax.GatherScatterMode.PROMISE_IN_BOUNDS)
                    for kk in range(D // 16):
                        sl = pl.ds(kk * 16, 16)
                        rows[i + jj, sl] = rows[i + jj, sl] * bex

        def _load_idx(t, srcbuf, dstbuf, exbuf):
            off = base + t * GCH
            pltpu.sync_copy(src_hbm.at[pl.ds(off, GCH)], srcbuf)
            pltpu.sync_copy(dst_hbm.at[pl.ds(off, GCH)], dstbuf)
            pltpu.sync_copy(ex_hbm.at[pl.ds(off, GCH)], exbuf)

        def _issue(srcbuf, rows, sem):
            pltpu.async_copy(tab_hbm.at[srcbuf], rows, sem)

        def _wait(srcbuf, rows, sem):
            pltpu.make_async_copy(tab_hbm.at[srcbuf], rows, sem).wait()

        _load_idx(0, src_a, dst_a, ex_a)
        _issue(src_a, rows_a, gsem_a)

        @pl.loop(0, NCH - 1, step=2)
        def _pair(t):
            _load_idx(t + 1, src_b, dst_b, ex_b)
            _issue(src_b, rows_b, gsem_b)
            _wait(src_a, rows_a, gsem_a)
            _scale(ex_a, rows_a)
            pltpu.sync_copy(rows_a, acc_sh.at[dst_a], add=True)
            _load_idx(t + 2, src_a, dst_a, ex_a)
            _issue(src_a, rows_a, gsem_a)
            _wait(src_b, rows_b, gsem_b)
            _scale(ex_b, rows_b)
            pltpu.sync_copy(rows_b, acc_sh.at[dst_b], add=True)

        _wait(src_a, rows_a, gsem_a)
        _scale(ex_a, rows_a)
        pltpu.sync_copy(rows_a, acc_sh.at[dst_a], add=True)

        plsc.subcore_barrier()

        @pl.loop(0, RB, step=GCH)
        def _out(r):
            pltpu.sync_copy(acc_sh.at[pl.ds(sid * RB + r, GCH)],
                            out_hbm.at[cid, pl.ds(sid * RB + r, GCH)])

    return k(table, src, dst, ex)


def _instance_norm_in_kernel(y, bm):
    oh = (bm == jax.lax.broadcasted_iota(jnp.int32, (NP, G), 1))
    oh = oh.astype(jnp.float32)
    ones = jnp.ones((NP, 1), jnp.float32)
    cnt = jax.lax.dot_general(oh, ones, (((0,), (0,)), ((), ())),
                              precision=_PREC,
                              preferred_element_type=jnp.float32)
    cnt = jnp.maximum(cnt, 1.0)
    sums = jax.lax.dot_general(oh, y, (((0,), (0,)), ((), ())),
                               precision=_PREC,
                               preferred_element_type=jnp.float32)
    mean = sums / cnt
    xc = y - _dot(oh, mean)
    sq = jax.lax.dot_general(oh, xc * xc, (((0,), (0,)), ((), ())),
                             precision=_PREC,
                             preferred_element_type=jnp.float32)
    var = sq / cnt
    return xc / jnp.sqrt(_dot(oh, var) + EPS)


def _comb_body(parts_ref, den_ref, y_ref):
    den = den_ref[0, :] + den_ref[1, :] + 1e-16
    y_ref[...] = (parts_ref[0] + parts_ref[1]) / den[:, None]


def _norm_only_body(z_ref, bm_ref, o_ref):
    o_ref[...] = _instance_norm_in_kernel(z_ref[...], bm_ref[...])


def _norm0(parts, den2, bm_pad):
    y = pl.pallas_call(
        _comb_body,
        out_shape=jax.ShapeDtypeStruct((NP, H), jnp.float32),
    )(parts, den2)
    return pl.pallas_call(
        _norm_only_body,
        out_shape=jax.ShapeDtypeStruct((NP, H), jnp.float32),
    )(y, bm_pad)


def _gc_body(parts_ref, den_ref, xp_ref, wn_ref, wr_ref, b_ref, z_ref):
    den = den_ref[0, :] + den_ref[1, :] + 1e-16
    agg = (parts_ref[0] + parts_ref[1]) / den[:, None]
    z_ref[...] = _dot(agg, wn_ref[...]) + _dot(xp_ref[...], wr_ref[...]) \
        + b_ref[0, :][None, :]


def _norm12(parts, den2, bm_pad, x_prev, wn, wr, b):
    z = pl.pallas_call(
        _gc_body,
        out_shape=jax.ShapeDtypeStruct((NP, H), jnp.float32),
    )(parts, den2, x_prev, wn, wr, b.reshape(1, H))
    return pl.pallas_call(
        _norm_only_body,
        out_shape=jax.ShapeDtypeStruct((NP, H), jnp.float32),
    )(z, bm_pad)


def kernel(x, ei, ea, batch_mask, W_egat, a_src, a_dst, a_edge,
           W1_root, W1_nbr, b1, W2_root, W2_nbr, b2):
    src, dst = ei[0], ei[1]

    h, s2, eaa = _pre(x, W_egat, a_src, a_dst, ea, a_edge)
    ex, den2 = _edge_softmax_sc(s2, src, dst, eaa.reshape(-1))

    bm_pad = jnp.pad(batch_mask, (0, NP - N),
                     constant_values=G).reshape(NP, 1)

    parts0 = _spmm_sc(h, src, dst, ex)
    x0 = _norm0(parts0, den2, bm_pad)

    wn1 = W1_nbr[:H] + W1_nbr[H:]
    wr1 = W1_root[:H] + W1_root[H:]
    parts1 = _spmm_sc(x0, src, dst, ex)
    x1 = _norm12(parts1, den2, bm_pad, x0, wn1, wr1, b1)

    wn2 = W2_nbr[:H] + W2_nbr[H:]
    wr2 = W2_root[:H] + W2_root[H:]
    parts2 = _spmm_sc(x1, src, dst, ex)
    x2 = _norm12(parts2, den2, bm_pad, x1, wn2, wr2, b2)

    return (x0[:N], x1[:N], x2[:N])


# speedup vs baseline: 16.4647x; 1.1828x over previous
import dataclasses
import functools

import jax
import jax.numpy as jnp
from jax import lax
from jax.experimental import pallas as pl
from jax.experimental.pallas import tpu as pltpu
from jax.experimental.pallas import tpu_sc as plsc

N = 10000
E = 320000
D = 128
H = 128
G = 64
EPS = 1e-5

NP = 10240
RB = NP // 16
ECH = 2000
EPT = E // 32
GCH = 80
NCH_REAL = (E // 2) // 16 // GCH
NCH = 126

_PREC = jax.lax.Precision.HIGHEST

_MESH = plsc.VectorSubcoreMesh(core_axis_name="c", subcore_axis_name="s",
                               num_cores=2, num_subcores=16)

_SC_CP = pltpu.CompilerParams()
if "needs_layout_passes" in pltpu.CompilerParams.__dataclass_fields__:
    _SC_CP = dataclasses.replace(_SC_CP, needs_layout_passes=False)


def _dot(a, b):
    return jax.lax.dot_general(a, b, (((1,), (0,)), ((), ())),
                               precision=_PREC,
                               preferred_element_type=jnp.float32)


def _pre_body(x_ref, w_ref, asrc_ref, adst_ref, ea_ref, aedge_ref,
              h_ref, s_ref, eaa_ref):
    x = x_ref[...]
    h = _dot(x, w_ref[...])
    h_ref[...] = h
    s_ref[0, :] = jnp.sum(h * asrc_ref[0, :][None, :], axis=1)
    s_ref[1, :] = jnp.sum(h * adst_ref[0, :][None, :], axis=1)
    eaa_ref[...] = ea_ref[...] * aedge_ref[0, 0]


def _pre(x, w, a_src, a_dst, ea, a_edge):
    return pl.pallas_call(
        _pre_body,
        out_shape=[
            jax.ShapeDtypeStruct((N, D), jnp.float32),
            jax.ShapeDtypeStruct((2, N), jnp.float32),
            jax.ShapeDtypeStruct((E // 128, 128), jnp.float32),
        ],
    )(x, w, a_src.reshape(1, H), a_dst.reshape(1, H),
      ea.reshape(E // 128, 128), a_edge.reshape(1, 1))


def _edge_softmax_sc(s2, src, dst, eaa):
    @functools.partial(
        pl.kernel,
        out_type=[jax.ShapeDtypeStruct((E,), jnp.float32),
                  jax.ShapeDtypeStruct((2, NP), jnp.float32)],
        mesh=_MESH,
        compiler_params=_SC_CP,
        scratch_types=[pltpu.VMEM((N,), jnp.float32),
                       pltpu.VMEM((N,), jnp.float32),
                       pltpu.VMEM((ECH,), jnp.int32),
                       pltpu.VMEM((ECH,), jnp.int32),
                       pltpu.VMEM((ECH,), jnp.float32),
                       pltpu.VMEM((ECH,), jnp.float32),
                       pltpu.VMEM((NP,), jnp.float32),
                       pltpu.VMEM((RB,), jnp.float32),
                       pltpu.VMEM((RB,), jnp.float32),
                       pltpu.VMEM_SHARED((16, NP), jnp.float32)],
    )
    def k(s_hbm, src_hbm, dst_hbm, eaa_hbm, ex_hbm, den_hbm,
          ssrc_v, sdst_v, src_v, dst_v, eaa_v, ex_v, den_v, comb_v, tmp_v,
          stage_sh):
        cid = lax.axis_index("c")
        sid = lax.axis_index("s")
        base = (cid * 16 + sid) * EPT
        pltpu.sync_copy(s_hbm.at[0], ssrc_v)
        pltpu.sync_copy(s_hbm.at[1], sdst_v)

        @pl.loop(0, NP, step=16)
        def _z(i):
            den_v[pl.ds(i, 16)] = jnp.zeros((16,), jnp.float32)

        @pl.loop(0, EPT, step=ECH)
        def _chunk(off):
            pltpu.sync_copy(src_hbm.at[pl.ds(base + off, ECH)], src_v)
            pltpu.sync_copy(dst_hbm.at[pl.ds(base + off, ECH)], dst_v)
            pltpu.sync_copy(eaa_hbm.at[pl.ds(base + off, ECH)], eaa_v)

            @pl.loop(0, ECH, step=16)
            def _grp(i):
                si = src_v[pl.ds(i, 16)]
                di = dst_v[pl.ds(i, 16)]
                a = plsc.load_gather(ssrc_v, [si])
                b = plsc.load_gather(sdst_v, [di])
                lg = a + b + eaa_v[pl.ds(i, 16)]
                e = jnp.maximum(lg, 0.2 * lg)
                exv = jnp.exp(e)
                ex_v[pl.ds(i, 16)] = exv
                plsc.addupdate_scatter(den_v, [di], exv)

            pltpu.sync_copy(ex_v, ex_hbm.at[pl.ds(base + off, ECH)])

        pltpu.sync_copy(den_v, stage_sh.at[sid])
        plsc.subcore_barrier()
        pltpu.sync_copy(stage_sh.at[0, pl.ds(sid * RB, RB)], comb_v)

        @pl.loop(1, 16)
        def _red(j):
            pltpu.sync_copy(stage_sh.at[j, pl.ds(sid * RB, RB)], tmp_v)

            @pl.loop(0, RB, step=16)
            def _add(i):
                comb_v[pl.ds(i, 16)] = (comb_v[pl.ds(i, 16)]
                                        + tmp_v[pl.ds(i, 16)])

        pltpu.sync_copy(comb_v, den_hbm.at[cid, pl.ds(sid * RB, RB)])

    return k(s2, src, dst, eaa)


def _spmm_sc(table, packed):
    @functools.partial(
        pl.kernel,
        out_type=jax.ShapeDtypeStruct((2, NP, D), jnp.float32),
        mesh=_MESH,
        compiler_params=_SC_CP,
        scratch_types=[pltpu.VMEM((3, GCH), jnp.int32),
                       pltpu.VMEM((3, GCH), jnp.int32),
                       pltpu.VMEM((3, GCH), jnp.int32),
                       pltpu.VMEM((GCH, D), jnp.float32),
                       pltpu.VMEM((GCH, D), jnp.float32),
                       pltpu.VMEM((GCH, D), jnp.float32),
                       pltpu.VMEM_SHARED((NP, D), jnp.float32),
                       pltpu.SemaphoreType.DMA,
                       pltpu.SemaphoreType.DMA,
                       pltpu.SemaphoreType.DMA,
                       pltpu.SemaphoreType.DMA,
                       pltpu.SemaphoreType.DMA,
                       pltpu.SemaphoreType.DMA,
                       pltpu.SemaphoreType.DMA,
                       pltpu.SemaphoreType.DMA,
                       pltpu.SemaphoreType.DMA],
    )
    def k(tab_hbm, pk_hbm, out_hbm,
          idx0, idx1, idx2, rows0, rows1, rows2, acc_sh,
          isem0, isem1, isem2, gsem0, gsem1, gsem2, ssem0, ssem1, ssem2):
        cid = lax.axis_index("c")
        sid = lax.axis_index("s")
        cbase = (cid * 16 + sid) * NCH

        idxs = (idx0, idx1, idx2)
        rows = (rows0, rows1, rows2)
        isems = (isem0, isem1, isem2)
        gsems = (gsem0, gsem1, gsem2)
        ssems = (ssem0, ssem1, ssem2)

        @pl.loop(0, GCH)
        def _zr(r):
            for kk in range(D // 16):
                rows0[r, pl.ds(kk * 16, 16)] = jnp.zeros((16,), jnp.float32)

        @pl.loop(0, RB, step=GCH)
        def _za(r):
            pltpu.sync_copy(rows0, acc_sh.at[pl.ds(sid * RB + r, GCH)])

        plsc.subcore_barrier()

        _gdn = jax.lax.GatherDimensionNumbers(
            offset_dims=(), collapsed_slice_dims=(0,), start_index_map=(0,))

        def _scale(s):
            @pl.loop(0, GCH, step=16)
            def _g(i):
                ex16 = plsc.bitcast(idxs[s][2, pl.ds(i, 16)], jnp.float32)
                for jj in range(16):
                    bex = jax.lax.gather(
                        ex16, jnp.full((16, 1), jj, jnp.int32), _gdn, (1,),
                        mode=jax.lax.GatherScatterMode.PROMISE_IN_BOUNDS)
                    for kk in range(D // 16):
                        sl = pl.ds(kk * 16, 16)
                        rows[s][i + jj, sl] = rows[s][i + jj, sl] * bex

        def _lidx(t, s):
            pltpu.async_copy(pk_hbm.at[cbase + t], idxs[s], isems[s])

        def _gather(s):
            pltpu.async_copy(tab_hbm.at[idxs[s].at[0]], rows[s], gsems[s])

        def _scatter(s):
            pltpu.async_copy(rows[s], acc_sh.at[idxs[s].at[1]], ssems[s],
                             add=True)

        def _wait_i(s):
            pltpu.make_async_copy(pk_hbm.at[cbase], idxs[s], isems[s]).wait()

        def _wait_g(s):
            pltpu.make_async_copy(tab_hbm.at[idxs[s].at[0]], rows[s],
                                  gsems[s]).wait()

        def _wait_s(s):
            pltpu.make_async_copy(rows[s], acc_sh.at[idxs[s].at[1]],
                                  ssems[s]).wait()

        _lidx(0, 0)
        _lidx(1, 1)
        _lidx(2, 2)
        _wait_i(0)
        _gather(0)
        _wait_i(1)
        _gather(1)

        @pl.loop(0, NCH, step=3)
        def _ring(t):
            for kpos in range(3):
                s = kpos
                s2 = (kpos + 2) % 3
                tc = t + kpos
                _wait_g(s)
                _scale(s)
                _scatter(s)

                @pl.when(tc + 3 < NCH)
                def _pf():
                    _lidx(tc + 3, s)

                @pl.when(tc + 2 < NCH)
                def _nx():
                    _wait_i(s2)

                    @pl.when(tc >= 1)
                    def _ws():
                        _wait_s(s2)

                    _gather(s2)

        _wait_s(0)
        _wait_s(1)
        _wait_s(2)

        plsc.subcore_barrier()

        @pl.loop(0, RB, step=GCH)
        def _out(r):
            pltpu.sync_copy(acc_sh.at[pl.ds(sid * RB + r, GCH)],
                            out_hbm.at[cid, pl.ds(sid * RB + r, GCH)])

    return k(table, packed)


def _instance_norm_in_kernel(y, bm):
    oh = (bm == jax.lax.broadcasted_iota(jnp.int32, (NP, G), 1))
    oh = oh.astype(jnp.float32)
    ones = jnp.ones((NP, 1), jnp.float32)
    cnt = jax.lax.dot_general(oh, ones, (((0,), (0,)), ((), ())),
                              precision=_PREC,
                              preferred_element_type=jnp.float32)
    cnt = jnp.maximum(cnt, 1.0)
    sums = jax.lax.dot_general(oh, y, (((0,), (0,)), ((), ())),
                               precision=_PREC,
                               preferred_element_type=jnp.float32)
    mean = sums / cnt
    xc = y - _dot(oh, mean)
    sq = jax.lax.dot_general(oh, xc * xc, (((0,), (0,)), ((), ())),
                             precision=_PREC,
                             preferred_element_type=jnp.float32)
    var = sq / cnt
    return xc / jnp.sqrt(_dot(oh, var) + EPS)


def _comb_body(parts_ref, den_ref, y_ref):
    den = den_ref[0, :] + den_ref[1, :] + 1e-16
    y_ref[...] = (parts_ref[0] + parts_ref[1]) / den[:, None]


def _norm_only_body(z_ref, bm_ref, o_ref):
    o_ref[...] = _instance_norm_in_kernel(z_ref[...], bm_ref[...])


def _norm0(parts, den2, bm_pad):
    y = pl.pallas_call(
        _comb_body,
        out_shape=jax.ShapeDtypeStruct((NP, H), jnp.float32),
    )(parts, den2)
    return pl.pallas_call(
        _norm_only_body,
        out_shape=jax.ShapeDtypeStruct((NP, H), jnp.float32),
    )(y, bm_pad)


def _gc_body(parts_ref, den_ref, xp_ref, wn_ref, wr_ref, b_ref, z_ref):
    den = den_ref[0, :] + den_ref[1, :] + 1e-16
    agg = (parts_ref[0] + parts_ref[1]) / den[:, None]
    z_ref[...] = _dot(agg, wn_ref[...]) + _dot(xp_ref[...], wr_ref[...]) \
        + b_ref[0, :][None, :]


def _norm12(parts, den2, bm_pad, x_prev, wn, wr, b):
    z = pl.pallas_call(
        _gc_body,
        out_shape=jax.ShapeDtypeStruct((NP, H), jnp.float32),
    )(parts, den2, x_prev, wn, wr, b.reshape(1, H))
    return pl.pallas_call(
        _norm_only_body,
        out_shape=jax.ShapeDtypeStruct((NP, H), jnp.float32),
    )(z, bm_pad)


def kernel(x, ei, ea, batch_mask, W_egat, a_src, a_dst, a_edge,
           W1_root, W1_nbr, b1, W2_root, W2_nbr, b2):
    src, dst = ei[0], ei[1]

    h, s2, eaa = _pre(x, W_egat, a_src, a_dst, ea, a_edge)
    ex, den2 = _edge_softmax_sc(s2, src, dst, eaa.reshape(-1))

    bm_pad = jnp.pad(batch_mask, (0, NP - N),
                     constant_values=G).reshape(NP, 1)

    exbits = jax.lax.bitcast_convert_type(ex, jnp.int32)
    stack = jnp.stack([src.reshape(32, NCH_REAL, GCH),
                       dst.reshape(32, NCH_REAL, GCH),
                       exbits.reshape(32, NCH_REAL, GCH)], axis=2)
    pad = jnp.zeros((32, NCH - NCH_REAL, 3, GCH), jnp.int32)
    packed = jnp.concatenate([stack, pad], axis=1).reshape(32 * NCH, 3, GCH)

    parts0 = _spmm_sc(h, packed)
    x0 = _norm0(parts0, den2, bm_pad)

    wn1 = W1_nbr[:H] + W1_nbr[H:]
    wr1 = W1_root[:H] + W1_root[H:]
    parts1 = _spmm_sc(x0, packed)
    x1 = _norm12(parts1, den2, bm_pad, x0, wn1, wr1, b1)

    wn2 = W2_nbr[:H] + W2_nbr[H:]
    wr2 = W2_root[:H] + W2_root[H:]
    parts2 = _spmm_sc(x1, packed)
    x2 = _norm12(parts2, den2, bm_pad, x1, wn2, wr2, b2)

    return (x0[:N], x1[:N], x2[:N])
